# Initial kernel scaffold; baseline (speedup 1.0000x reference)
import jax
import jax.numpy as jnp
from jax.experimental import pallas as pl


def kernel(x, table):
    raise NotImplementedError("write your pallas kernel here")


# speedup vs baseline: 1.1030x; 1.1030x over previous
import functools

import jax
import jax.numpy as jnp
from jax import lax
from jax.experimental import pallas as pl
from jax.experimental.pallas import tpu as pltpu
from jax.experimental.pallas import tpu_sc as plsc

EMB = 32
NW = 32
GRAN = 128
BUF_G = 8
BUF_ROWS = GRAN * BUF_G


def _make_kernel(n_rows):
    rows_per_w = n_rows // NW
    g_per_w = rows_per_w // GRAN
    n_iters = g_per_w // BUF_G
    mesh = plsc.VectorSubcoreMesh(core_axis_name="c", subcore_axis_name="s")

    @functools.partial(
        pl.kernel,
        mesh=mesh,
        out_type=jax.ShapeDtypeStruct((n_rows, EMB), jnp.float32),
        scratch_types=[
            pltpu.VMEM((g_per_w, GRAN), jnp.int32),
            pltpu.VMEM((BUF_ROWS, EMB), jnp.float32),
            pltpu.SemaphoreType.DMA,
        ],
        compiler_params=pltpu.CompilerParams(use_tc_tiling_on_sc=False),
    )
    def k(x_hbm, table_hbm, out_hbm, idx_v, rows_v, sem):
        wid = lax.axis_index("s") * 2 + lax.axis_index("c")
        base = wid * rows_per_w
        pltpu.sync_copy(x_hbm.at[wid], idx_v)

        def body(i, carry):
            descs = []
            for j in range(BUF_G):
                d = pltpu.async_copy(
                    table_hbm.at[idx_v.at[i * BUF_G + j]],
                    rows_v.at[pl.ds(j * GRAN, GRAN)],
                    sem,
                )
                descs.append(d)
            for d in descs:
                d.wait()
            pltpu.sync_copy(
                rows_v, out_hbm.at[pl.ds(base + i * BUF_ROWS, BUF_ROWS)]
            )
            return carry

        lax.fori_loop(0, n_iters, body, 0)

    return k


@jax.jit
def kernel(x, table):
    orig_shape = x.shape
    n = x.size
    x_flat = x.reshape(NW, (n // NW) // GRAN, GRAN).astype(jnp.int32)
    out = _make_kernel(n)(x_flat, table)
    return out.reshape(*orig_shape, EMB)


# speedup vs baseline: 1.1135x; 1.0095x over previous
import functools

import jax
import jax.numpy as jnp
from jax import lax
from jax.experimental import pallas as pl
from jax.experimental.pallas import tpu as pltpu
from jax.experimental.pallas import tpu_sc as plsc

EMB = 32
NW = 32
GRAN = 128
NBUF = 4
BUF_G = 5
CHUNK = GRAN * BUF_G


def _make_kernel(n_rows):
    rows_per_w = n_rows // NW
    g_per_w = rows_per_w // GRAN
    n_chunks = g_per_w // BUF_G
    n_super = n_chunks // NBUF
    mesh = plsc.VectorSubcoreMesh(core_axis_name="c", subcore_axis_name="s")

    @functools.partial(
        pl.kernel,
        mesh=mesh,
        out_type=jax.ShapeDtypeStruct((n_rows, EMB), jnp.float32),
        scratch_types=[
            pltpu.VMEM((g_per_w, GRAN), jnp.int32),
            [pltpu.VMEM((CHUNK, EMB), jnp.float32) for _ in range(NBUF)],
            [pltpu.SemaphoreType.DMA for _ in range(NBUF)],
        ],
        compiler_params=pltpu.CompilerParams(use_tc_tiling_on_sc=False),
    )
    def k(x_hbm, table_hbm, out_hbm, idx_v, bufs, sems):
        wid = lax.axis_index("s") * 2 + lax.axis_index("c")
        base = wid * rows_per_w
        pltpu.sync_copy(x_hbm.at[wid], idx_v)

        def fire(b, c):
            for j in range(BUF_G):
                pltpu.async_copy(
                    table_hbm.at[idx_v.at[c * BUF_G + j]],
                    bufs[b].at[pl.ds(j * GRAN, GRAN)],
                    sems[b],
                )

        for b in range(NBUF):
            fire(b, b)

        def body(s, carry):
            for b in range(NBUF):
                c = s * NBUF + b
                pltpu.make_async_copy(
                    out_hbm.at[pl.ds(0, CHUNK)], bufs[b], sems[b]
                ).wait()
                pltpu.sync_copy(
                    bufs[b], out_hbm.at[pl.ds(base + c * CHUNK, CHUNK)]
                )

                @pl.when(s < n_super - 1)
                def _():
                    fire(b, c + NBUF)

            return carry

        lax.fori_loop(0, n_super, body, 0)

    return k


@jax.jit
def kernel(x, table):
    orig_shape = x.shape
    n = x.size
    x_flat = x.reshape(NW, (n // NW) // GRAN, GRAN).astype(jnp.int32)
    out = _make_kernel(n)(x_flat, table)
    return out.reshape(*orig_shape, EMB)


# speedup vs baseline: 1.1135x; 1.0000x over previous
import functools

import jax
import jax.numpy as jnp
from jax import lax
from jax.experimental import pallas as pl
from jax.experimental.pallas import tpu as pltpu
from jax.experimental.pallas import tpu_sc as plsc

EMB = 32
NW = 32
GRAN = 640
NBUF = 4
BUF_G = 1
CHUNK = GRAN * BUF_G


def _make_kernel(n_rows):
    rows_per_w = n_rows // NW
    g_per_w = rows_per_w // GRAN
    n_chunks = g_per_w // BUF_G
    n_super = n_chunks // NBUF
    mesh = plsc.VectorSubcoreMesh(core_axis_name="c", subcore_axis_name="s")

    @functools.partial(
        pl.kernel,
        mesh=mesh,
        out_type=jax.ShapeDtypeStruct((n_rows, EMB), jnp.float32),
        scratch_types=[
            pltpu.VMEM((g_per_w, GRAN), jnp.int32),
            [pltpu.VMEM((CHUNK, EMB), jnp.float32) for _ in range(NBUF)],
            [pltpu.SemaphoreType.DMA for _ in range(NBUF)],
        ],
        compiler_params=pltpu.CompilerParams(use_tc_tiling_on_sc=False),
    )
    def k(x_hbm, table_hbm, out_hbm, idx_v, bufs, sems):
        wid = lax.axis_index("s") * 2 + lax.axis_index("c")
        base = wid * rows_per_w
        pltpu.sync_copy(x_hbm.at[wid], idx_v)

        def fire(b, c):
            for j in range(BUF_G):
                pltpu.async_copy(
                    table_hbm.at[idx_v.at[c * BUF_G + j]],
                    bufs[b].at[pl.ds(j * GRAN, GRAN)],
                    sems[b],
                )

        for b in range(NBUF):
            fire(b, b)

        def body(s, carry):
            for b in range(NBUF):
                c = s * NBUF + b
                pltpu.make_async_copy(
                    out_hbm.at[pl.ds(0, CHUNK)], bufs[b], sems[b]
                ).wait()
                pltpu.sync_copy(
                    bufs[b], out_hbm.at[pl.ds(base + c * CHUNK, CHUNK)]
                )

                @pl.when(s < n_super - 1)
                def _():
                    fire(b, c + NBUF)

            return carry

        lax.fori_loop(0, n_super, body, 0)

    return k


@jax.jit
def kernel(x, table):
    orig_shape = x.shape
    n = x.size
    x_flat = x.reshape(NW, (n // NW) // GRAN, GRAN).astype(jnp.int32)
    out = _make_kernel(n)(x_flat, table)
    return out.reshape(*orig_shape, EMB)


# speedup vs baseline: 1.6504x; 1.4821x over previous
import functools

import jax
import jax.numpy as jnp
from jax import lax
from jax.experimental import pallas as pl
from jax.experimental.pallas import tpu as pltpu
from jax.experimental.pallas import tpu_sc as plsc

EMB = 32
NW = 32
GRAN = 128
NBUF = 4
S = 50
B = 16384
NG = S * (B // GRAN)
G_PER_W = NG // NW
JJ = B // GRAN
N_SUPER = G_PER_W // NBUF


def _make_kernel():
    mesh = plsc.VectorSubcoreMesh(core_axis_name="c", subcore_axis_name="s")

    @functools.partial(
        pl.kernel,
        mesh=mesh,
        out_type=jax.ShapeDtypeStruct((S, EMB // 8, JJ, 8 * GRAN), jnp.float32),
        scratch_types=[
            pltpu.VMEM((G_PER_W, GRAN), jnp.int32),
            [pltpu.VMEM((GRAN, EMB), jnp.float32) for _ in range(NBUF)],
            [pltpu.VMEM((EMB // 8, 8 * GRAN), jnp.float32) for _ in range(2)],
            [pltpu.SemaphoreType.DMA for _ in range(NBUF)],
            [pltpu.SemaphoreType.DMA for _ in range(2)],
        ],
        compiler_params=pltpu.CompilerParams(
            use_tc_tiling_on_sc=False, needs_layout_passes=False
        ),
    )
    def k(x_hbm, table_hbm, z_hbm, idx_v, gbufs, zbufs, gsems, zsems):
        wid = lax.axis_index("s") * 2 + lax.axis_index("c")
        g0 = wid * G_PER_W
        pltpu.sync_copy(x_hbm.at[pl.ds(g0, G_PER_W)], idx_v)

        cvecs = [cb * 16 + lax.iota(jnp.int32, 16) for cb in range(8)]

        def fire(b, t):
            pltpu.async_copy(table_hbm.at[idx_v.at[t]], gbufs[b], gsems[b])

        for b in range(NBUF):
            fire(b, b)

        def body(u, carry):
            for b in range(NBUF):
                t = u * NBUF + b
                p = b % 2
                g = g0 + t
                s = g // JJ
                jj = lax.rem(g, JJ)
                pltpu.make_async_copy(
                    table_hbm.at[pl.ds(0, GRAN)], gbufs[b], gsems[b]
                ).wait()

                def zwait():
                    pltpu.make_async_copy(
                        zbufs[p], z_hbm.at[0, :, 0], zsems[p]
                    ).wait()

                if b < 2:

                    @pl.when(u >= 1)
                    def _():
                        zwait()

                else:
                    zwait()

                def transpose_d(d, c):
                    dvec = jnp.full((16,), d, jnp.int32)
                    row = d // 8
                    off = lax.rem(d, 8) * GRAN
                    for cb in range(8):
                        v = plsc.load_gather(gbufs[b], [cvecs[cb], dvec])
                        zbufs[p][row, pl.ds(off + cb * 16, 16)] = v
                    return c

                lax.fori_loop(0, EMB, transpose_d, 0)

                pltpu.async_copy(zbufs[p], z_hbm.at[s, :, jj], zsems[p])

                @pl.when(u < N_SUPER - 1)
                def _():
                    fire(b, t + NBUF)

            return carry

        lax.fori_loop(0, N_SUPER, body, 0)
        for p in range(2):
            pltpu.make_async_copy(zbufs[p], z_hbm.at[0, :, 0], zsems[p]).wait()

    return k


@jax.jit
def kernel(x, table):
    x4 = x.T.reshape(NG, GRAN).astype(jnp.int32)
    z = _make_kernel()(x4, table)
    return (
        z.reshape(S, EMB // 8, JJ, 8, GRAN)
        .transpose(2, 4, 0, 1, 3)
        .reshape(B, S, EMB)
    )
